# Initial kernel scaffold; baseline (speedup 1.0000x reference)
import jax
import jax.numpy as jnp
from jax.experimental import pallas as pl


def kernel(x, edge_index, edge_to_node_index, edge_to_node, routing_table_item, hx, cx, W1l, b1, W1r, W2l, b2, W2r, Wih, bih, Whh, bhh, W0, b0, Wa, ba, Wb, bb, Wc, bc):
    raise NotImplementedError("write your pallas kernel here")


# speedup vs baseline: 5.6195x; 5.6195x over previous
import jax
import jax.numpy as jnp
from jax import lax
from jax.experimental import pallas as pl
from jax.experimental.pallas import tpu as pltpu

_F32 = jnp.float32
_HI = lax.Precision.HIGHEST


def _dot(a, b):
    return jnp.dot(a, b, precision=_HI, preferred_element_type=_F32)


def _net_body(rti_ref, src_ref, dst_ref, e2ni_ref, e2ns_ref, e2nt_ref,
              x_ref, hx_ref, cx_ref,
              w1l_ref, b1_ref, w1r_ref, w2l_ref, b2_ref, w2r_ref,
              wih_ref, bih_ref, whh_ref, bhh_ref,
              w0_ref, b0_ref, wa_ref, ba_ref, wb_ref, bb_ref, wc_ref, bc_ref,
              out_ref, g_scratch):
    src_oh = (src_ref[...] == lax.broadcasted_iota(jnp.int32, (2048, 64), 1)
              ).astype(_F32)
    dst_oh_t = (dst_ref[...] == lax.broadcasted_iota(jnp.int32, (64, 2048), 0)
                ).astype(_F32)
    a_cnt = _dot(dst_oh_t, src_oh)
    inv_cnt = 1.0 / jnp.maximum(jnp.sum(a_cnt, axis=1, keepdims=True), 1.0)

    x = x_ref[...]
    agg1 = _dot(a_cnt, x) * inv_cnt
    h1 = jnp.maximum(agg1 * w1l_ref[...] + x * w1r_ref[...] + b1_ref[...], 0.0)

    agg2 = _dot(a_cnt, h1) * inv_cnt
    h2 = jnp.maximum(_dot(agg2, w2l_ref[...]) + _dot(h1, w2r_ref[...])
                     + b2_ref[...], 0.0)
    x_g = jnp.sum(h2, axis=0, keepdims=True) * (1.0 / 64.0)

    oh_idx = (e2ni_ref[...] == lax.broadcasted_iota(jnp.int32, (64, 128), 1)
              ).astype(_F32)
    v_iota = lax.broadcasted_iota(jnp.int32, (128, 64), 1)
    e2ns_oh = (e2ns_ref[...] == v_iota).astype(_F32)
    e2nt_oh = (e2nt_ref[...] == v_iota).astype(_F32)
    p0 = _dot(oh_idx, e2ns_oh)
    p1 = _dot(oh_idx, e2nt_oh)

    seq = jnp.concatenate([h2, p0, p1], axis=1)
    g_scratch[...] = _dot(seq, wih_ref[...]) + bih_ref[...]

    whh = whh_ref[...]
    bhh = bhh_ref[...]

    def step(t, carry):
        hh, cc = carry
        gates = g_scratch[pl.ds(t, 1), :] + _dot(hh, whh) + bhh
        i_g = jax.nn.sigmoid(gates[:, 0:64])
        f_g = jax.nn.sigmoid(gates[:, 64:128])
        g_g = jnp.tanh(gates[:, 128:192])
        o_g = jax.nn.sigmoid(gates[:, 192:256])
        cc = f_g * cc + i_g * g_g
        hh = o_g * jnp.tanh(cc)
        return hh, cc

    hh, cc = lax.fori_loop(0, 64, step, (hx_ref[...], cx_ref[...]))

    lane = lax.broadcasted_iota(jnp.int32, (1, 64), 1)
    s_oh = (lane == rti_ref[0]).astype(_F32)
    p_oh = (lane == rti_ref[1]).astype(_F32)
    d_oh = (lane == rti_ref[2]).astype(_F32)
    feat = jnp.concatenate([cc, hh, x_g, s_oh, p_oh, d_oh], axis=1)

    o = jnp.maximum(_dot(feat, w0_ref[...]) + b0_ref[...], 0.0)
    o = jnp.maximum(_dot(o, wa_ref[...]) + ba_ref[...], 0.0)
    o = jnp.maximum(_dot(o, wb_ref[...]) + bb_ref[...], 0.0)
    o = jnp.maximum(_dot(o, wc_ref[...]) + bc_ref[...], 0.0)
    out_ref[...] = o


def kernel(x, edge_index, edge_to_node_index, edge_to_node, routing_table_item,
           hx, cx, W1l, b1, W1r, W2l, b2, W2r, Wih, bih, Whh, bhh,
           W0, b0, Wa, ba, Wb, bb, Wc, bc):
    args = (
        routing_table_item,
        edge_index[0].reshape(2048, 1),
        edge_index[1].reshape(1, 2048),
        edge_to_node_index.reshape(64, 1),
        edge_to_node[:, 0:1],
        edge_to_node[:, 1:2],
        x,
        hx.reshape(1, 64),
        cx.reshape(1, 64),
        W1l, b1.reshape(1, 16), W1r,
        W2l, b2.reshape(1, 64), W2r,
        Wih, bih.reshape(1, 256), Whh, bhh.reshape(1, 256),
        W0, b0.reshape(1, 32), Wa, ba.reshape(1, 16),
        Wb, bb.reshape(1, 8), Wc, bc.reshape(1, 1),
    )
    in_specs = ([pl.BlockSpec(memory_space=pltpu.SMEM)]
                + [pl.BlockSpec(memory_space=pltpu.VMEM)] * (len(args) - 1))
    out = pl.pallas_call(
        _net_body,
        out_shape=jax.ShapeDtypeStruct((1, 1), jnp.float32),
        in_specs=in_specs,
        out_specs=pl.BlockSpec(memory_space=pltpu.VMEM),
        scratch_shapes=[pltpu.VMEM((64, 256), jnp.float32)],
    )(*args)
    return out.reshape(1)


# speedup vs baseline: 8.0280x; 1.4286x over previous
import jax
import jax.numpy as jnp
from jax import lax
from jax.experimental import pallas as pl
from jax.experimental.pallas import tpu as pltpu

_F32 = jnp.float32
_BF16 = jnp.bfloat16
_HI = lax.Precision.HIGHEST


def _dot(a, b):
    return jnp.dot(a, b, precision=_HI, preferred_element_type=_F32)


def _sigmoid(x):
    return 0.5 + 0.5 * jnp.tanh(0.5 * x)


def _net_body(rti_ref, src_ref, dst_ref, e2ni_ref, e2ns_ref, e2nt_ref,
              x_ref, hx_ref, cx_ref,
              w1l_ref, b1_ref, w1r_ref, w2l_ref, b2_ref, w2r_ref,
              wih_i_ref, wih_f_ref, wih_g_ref, wih_o_ref, bihh_ref,
              whh_i_ref, whh_f_ref, whh_g_ref, whh_o_ref,
              w0_ref, b0_ref, wa_ref, ba_ref, wb_ref, bb_ref, wc_ref, bc_ref,
              out_ref, gi_s, gf_s, gg_s, go_s):
    src_oh = (src_ref[...] == lax.broadcasted_iota(jnp.int32, (2048, 64), 1)
              ).astype(_BF16)
    dst_oh_t = (dst_ref[...] == lax.broadcasted_iota(jnp.int32, (64, 2048), 0)
                ).astype(_BF16)
    a_cnt = jnp.dot(dst_oh_t, src_oh, preferred_element_type=_F32)
    inv_cnt = 1.0 / jnp.maximum(jnp.sum(a_cnt, axis=1, keepdims=True), 1.0)

    x = x_ref[...]
    agg1 = _dot(a_cnt, x) * inv_cnt
    h1 = jnp.maximum(agg1 * w1l_ref[...] + x * w1r_ref[...] + b1_ref[...], 0.0)

    agg2 = _dot(a_cnt, h1) * inv_cnt
    h2 = jnp.maximum(_dot(agg2, w2l_ref[...]) + _dot(h1, w2r_ref[...])
                     + b2_ref[...], 0.0)
    x_g = jnp.sum(h2, axis=0, keepdims=True) * (1.0 / 64.0)

    oh_idx = (e2ni_ref[...] == lax.broadcasted_iota(jnp.int32, (64, 128), 1)
              ).astype(_BF16)
    v_iota = lax.broadcasted_iota(jnp.int32, (128, 64), 1)
    e2ns_oh = (e2ns_ref[...] == v_iota).astype(_BF16)
    e2nt_oh = (e2nt_ref[...] == v_iota).astype(_BF16)
    p0 = jnp.dot(oh_idx, e2ns_oh, preferred_element_type=_F32)
    p1 = jnp.dot(oh_idx, e2nt_oh, preferred_element_type=_F32)

    seq = jnp.concatenate([h2, p0, p1], axis=1)
    bihh = bihh_ref[...]
    gi_s[...] = _dot(seq, wih_i_ref[...]) + bihh[0:1, :]
    gf_s[...] = _dot(seq, wih_f_ref[...]) + bihh[1:2, :]
    gg_s[...] = _dot(seq, wih_g_ref[...]) + bihh[2:3, :]
    go_s[...] = _dot(seq, wih_o_ref[...]) + bihh[3:4, :]

    whh_i = whh_i_ref[...]
    whh_f = whh_f_ref[...]
    whh_g = whh_g_ref[...]
    whh_o = whh_o_ref[...]
    hh = hx_ref[...]
    cc = cx_ref[...]
    for t in range(64):
        hh_c = hh.reshape(64, 1)
        ri = jnp.sum(hh_c * whh_i, axis=0, keepdims=True)
        rf = jnp.sum(hh_c * whh_f, axis=0, keepdims=True)
        rg = jnp.sum(hh_c * whh_g, axis=0, keepdims=True)
        ro = jnp.sum(hh_c * whh_o, axis=0, keepdims=True)
        i_t = _sigmoid(gi_s[t:t + 1, :] + ri)
        f_t = _sigmoid(gf_s[t:t + 1, :] + rf)
        o_t = _sigmoid(go_s[t:t + 1, :] + ro)
        g_t = jnp.tanh(gg_s[t:t + 1, :] + rg)
        cc = f_t * cc + i_t * g_t
        hh = o_t * jnp.tanh(cc)

    lane = lax.broadcasted_iota(jnp.int32, (1, 64), 1)
    s_oh = (lane == rti_ref[0]).astype(_F32)
    p_oh = (lane == rti_ref[1]).astype(_F32)
    d_oh = (lane == rti_ref[2]).astype(_F32)
    feat = jnp.concatenate([cc, hh, x_g, s_oh, p_oh, d_oh], axis=1)

    o = jnp.maximum(_dot(feat, w0_ref[...]) + b0_ref[...], 0.0)
    o = jnp.maximum(_dot(o, wa_ref[...]) + ba_ref[...], 0.0)
    o = jnp.maximum(_dot(o, wb_ref[...]) + bb_ref[...], 0.0)
    o = jnp.maximum(_dot(o, wc_ref[...]) + bc_ref[...], 0.0)
    out_ref[...] = o


def kernel(x, edge_index, edge_to_node_index, edge_to_node, routing_table_item,
           hx, cx, W1l, b1, W1r, W2l, b2, W2r, Wih, bih, Whh, bhh,
           W0, b0, Wa, ba, Wb, bb, Wc, bc):
    bihh = (bih + bhh).reshape(4, 64)
    args = (
        routing_table_item,
        edge_index[0].reshape(2048, 1),
        edge_index[1].reshape(1, 2048),
        edge_to_node_index.reshape(64, 1),
        edge_to_node[:, 0:1],
        edge_to_node[:, 1:2],
        x,
        hx.reshape(1, 64),
        cx.reshape(1, 64),
        W1l, b1.reshape(1, 16), W1r,
        W2l, b2.reshape(1, 64), W2r,
        Wih[:, 0:64], Wih[:, 64:128], Wih[:, 128:192], Wih[:, 192:256],
        bihh,
        Whh[:, 0:64], Whh[:, 64:128], Whh[:, 128:192], Whh[:, 192:256],
        W0, b0.reshape(1, 32), Wa, ba.reshape(1, 16),
        Wb, bb.reshape(1, 8), Wc, bc.reshape(1, 1),
    )
    in_specs = ([pl.BlockSpec(memory_space=pltpu.SMEM)]
                + [pl.BlockSpec(memory_space=pltpu.VMEM)] * (len(args) - 1))
    out = pl.pallas_call(
        _net_body,
        out_shape=jax.ShapeDtypeStruct((1, 1), jnp.float32),
        in_specs=in_specs,
        out_specs=pl.BlockSpec(memory_space=pltpu.VMEM),
        scratch_shapes=[pltpu.VMEM((64, 64), jnp.float32)] * 4,
    )(*args)
    return out.reshape(1)


# speedup vs baseline: 9.5266x; 1.1867x over previous
import jax
import jax.numpy as jnp
from jax import lax
from jax.experimental import pallas as pl
from jax.experimental.pallas import tpu as pltpu

_F32 = jnp.float32
_BF16 = jnp.bfloat16
_HI = lax.Precision.HIGHEST


def _dot(a, b):
    return jnp.dot(a, b, precision=_HI, preferred_element_type=_F32)


def _sigmoid(x):
    return 0.5 + 0.5 * jnp.tanh(0.5 * x)


def _net_body(rti_ref, edge_ref, e2ni_ref, e2n_ref,
              x_ref, hx_ref, cx_ref,
              w1l_ref, b1_ref, w1r_ref, w2l_ref, b2_ref, w2r_ref,
              wih_ref, bih_ref, whh_ref, bhh_ref,
              w0_ref, b0_ref, wa_ref, ba_ref, wb_ref, bb_ref, wc_ref, bc_ref,
              out_ref, gi_s, gf_s, gg_s, go_s):
    niota = lax.broadcasted_iota(jnp.int32, (64, 2048), 0)
    src_oh_t = (edge_ref[0:1, :] == niota).astype(_BF16)
    dst_oh_t = (edge_ref[1:2, :] == niota).astype(_BF16)
    a_cnt = lax.dot_general(dst_oh_t, src_oh_t,
                            (((1,), (1,)), ((), ())),
                            preferred_element_type=_F32)
    inv_cnt = 1.0 / jnp.maximum(jnp.sum(a_cnt, axis=1, keepdims=True), 1.0)

    x = x_ref[...]
    agg1 = _dot(a_cnt, x) * inv_cnt
    h1 = jnp.maximum(agg1 * w1l_ref[...] + x * w1r_ref[...] + b1_ref[...], 0.0)

    agg2 = _dot(a_cnt, h1) * inv_cnt
    h2 = jnp.maximum(_dot(agg2, w2l_ref[...]) + _dot(h1, w2r_ref[...])
                     + b2_ref[...], 0.0)
    x_g = jnp.sum(h2, axis=0, keepdims=True) * (1.0 / 64.0)

    oh_idx = (e2ni_ref[...].reshape(64, 1)
              == lax.broadcasted_iota(jnp.int32, (64, 128), 1)
              ).astype(_BF16)
    v_iota = lax.broadcasted_iota(jnp.int32, (128, 64), 1)
    e2n = e2n_ref[...]
    e2ns_oh = (e2n[:, 0:1] == v_iota).astype(_BF16)
    e2nt_oh = (e2n[:, 1:2] == v_iota).astype(_BF16)
    p0 = jnp.dot(oh_idx, e2ns_oh, preferred_element_type=_F32)
    p1 = jnp.dot(oh_idx, e2nt_oh, preferred_element_type=_F32)

    seq = jnp.concatenate([h2, p0, p1], axis=1)
    wih = wih_ref[...]
    bihh = bih_ref[...] + bhh_ref[...]
    gi_s[...] = _dot(seq, wih[:, 0:64]) + bihh[:, 0:64]
    gf_s[...] = _dot(seq, wih[:, 64:128]) + bihh[:, 64:128]
    gg_s[...] = _dot(seq, wih[:, 128:192]) + bihh[:, 128:192]
    go_s[...] = _dot(seq, wih[:, 192:256]) + bihh[:, 192:256]

    whh = whh_ref[...]
    whh_i = whh[:, 0:64]
    whh_f = whh[:, 64:128]
    whh_g = whh[:, 128:192]
    whh_o = whh[:, 192:256]
    hh = hx_ref[...]
    cc = cx_ref[...]
    for t in range(64):
        hh_c = hh.reshape(64, 1)
        ri = jnp.sum(hh_c * whh_i, axis=0, keepdims=True)
        rf = jnp.sum(hh_c * whh_f, axis=0, keepdims=True)
        rg = jnp.sum(hh_c * whh_g, axis=0, keepdims=True)
        ro = jnp.sum(hh_c * whh_o, axis=0, keepdims=True)
        i_t = _sigmoid(gi_s[t:t + 1, :] + ri)
        f_t = _sigmoid(gf_s[t:t + 1, :] + rf)
        o_t = _sigmoid(go_s[t:t + 1, :] + ro)
        g_t = jnp.tanh(gg_s[t:t + 1, :] + rg)
        cc = f_t * cc + i_t * g_t
        hh = o_t * jnp.tanh(cc)

    lane = lax.broadcasted_iota(jnp.int32, (1, 64), 1)
    s_oh = (lane == rti_ref[0]).astype(_F32)
    p_oh = (lane == rti_ref[1]).astype(_F32)
    d_oh = (lane == rti_ref[2]).astype(_F32)
    feat = jnp.concatenate([cc, hh, x_g, s_oh, p_oh, d_oh], axis=1)

    o = jnp.maximum(_dot(feat, w0_ref[...]) + b0_ref[...], 0.0)
    o = jnp.maximum(_dot(o, wa_ref[...]) + ba_ref[...], 0.0)
    o = jnp.maximum(_dot(o, wb_ref[...]) + bb_ref[...], 0.0)
    o = jnp.maximum(_dot(o, wc_ref[...]) + bc_ref[...], 0.0)
    out_ref[...] = o


def kernel(x, edge_index, edge_to_node_index, edge_to_node, routing_table_item,
           hx, cx, W1l, b1, W1r, W2l, b2, W2r, Wih, bih, Whh, bhh,
           W0, b0, Wa, ba, Wb, bb, Wc, bc):
    args = (
        routing_table_item,
        edge_index,
        edge_to_node_index.reshape(1, 64),
        edge_to_node,
        x,
        hx.reshape(1, 64),
        cx.reshape(1, 64),
        W1l, b1.reshape(1, 16), W1r,
        W2l, b2.reshape(1, 64), W2r,
        Wih, bih.reshape(1, 256), Whh, bhh.reshape(1, 256),
        W0, b0.reshape(1, 32), Wa, ba.reshape(1, 16),
        Wb, bb.reshape(1, 8), Wc, bc.reshape(1, 1),
    )
    in_specs = ([pl.BlockSpec(memory_space=pltpu.SMEM)]
                + [pl.BlockSpec(memory_space=pltpu.VMEM)] * (len(args) - 1))
    out = pl.pallas_call(
        _net_body,
        out_shape=jax.ShapeDtypeStruct((1, 1), jnp.float32),
        in_specs=in_specs,
        out_specs=pl.BlockSpec(memory_space=pltpu.VMEM),
        scratch_shapes=[pltpu.VMEM((64, 64), jnp.float32)] * 4,
    )(*args)
    return out.reshape(1)


# speedup vs baseline: 11.8200x; 1.2407x over previous
import jax
import jax.numpy as jnp
from jax import lax
from jax.experimental import pallas as pl
from jax.experimental.pallas import tpu as pltpu

_F32 = jnp.float32
_BF16 = jnp.bfloat16
_HI = lax.Precision.HIGHEST


def _dot(a, b):
    return jnp.dot(a, b, precision=_HI, preferred_element_type=_F32)


def _sigmoid(x):
    return 0.5 + 0.5 * jnp.tanh(0.5 * x)


def _net_body(rti_ref, edge_ref, e2ni_ref, e2n_ref,
              x_ref, hx_ref, cx_ref,
              w1l_ref, b1_ref, w1r_ref, w2l_ref, b2_ref, w2r_ref,
              wih_ref, bih_ref, whh_ref, bhh_ref,
              w0_ref, b0_ref, wa_ref, ba_ref, wb_ref, bb_ref, wc_ref, bc_ref,
              out_ref, gi_s, gf_s, gg_s, go_s, wi_s, wf_s, wg_s, wo_s):
    niota = lax.broadcasted_iota(jnp.int32, (64, 2048), 0)
    src_oh_t = (edge_ref[0:1, :] == niota).astype(_BF16)
    dst_oh_t = (edge_ref[1:2, :] == niota).astype(_BF16)
    a_cnt = lax.dot_general(dst_oh_t, src_oh_t,
                            (((1,), (1,)), ((), ())),
                            preferred_element_type=_F32)
    inv_cnt = 1.0 / jnp.maximum(jnp.sum(a_cnt, axis=1, keepdims=True), 1.0)

    x = x_ref[...]
    agg1 = _dot(a_cnt, x) * inv_cnt
    h1 = jnp.maximum(agg1 * w1l_ref[...] + x * w1r_ref[...] + b1_ref[...], 0.0)

    agg2 = _dot(a_cnt, h1) * inv_cnt
    h2 = jnp.maximum(_dot(agg2, w2l_ref[...]) + _dot(h1, w2r_ref[...])
                     + b2_ref[...], 0.0)
    x_g = jnp.sum(h2, axis=0, keepdims=True) * (1.0 / 64.0)

    oh_idx = (e2ni_ref[...].reshape(64, 1)
              == lax.broadcasted_iota(jnp.int32, (64, 128), 1)
              ).astype(_BF16)
    v_iota = lax.broadcasted_iota(jnp.int32, (128, 64), 1)
    e2n = e2n_ref[...]
    e2ns_oh = (e2n[:, 0:1] == v_iota).astype(_BF16)
    e2nt_oh = (e2n[:, 1:2] == v_iota).astype(_BF16)
    p0 = jnp.dot(oh_idx, e2ns_oh, preferred_element_type=_F32)
    p1 = jnp.dot(oh_idx, e2nt_oh, preferred_element_type=_F32)

    seq = jnp.concatenate([h2, p0, p1], axis=1)
    wih = wih_ref[...]
    bihh = bih_ref[...] + bhh_ref[...]
    gi_s[...] = _dot(seq, wih[:, 0:64]) + bihh[:, 0:64]
    gf_s[...] = _dot(seq, wih[:, 64:128]) + bihh[:, 64:128]
    gg_s[...] = _dot(seq, wih[:, 128:192]) + bihh[:, 128:192]
    go_s[...] = _dot(seq, wih[:, 192:256]) + bihh[:, 192:256]

    whh = whh_ref[...]
    wi_s[...] = whh[:, 0:64]
    wf_s[...] = whh[:, 64:128]
    wg_s[...] = whh[:, 128:192]
    wo_s[...] = whh[:, 192:256]
    whh_i = wi_s[...]
    whh_f = wf_s[...]
    whh_g = wg_s[...]
    whh_o = wo_s[...]
    hh = hx_ref[...]
    cc = cx_ref[...]
    for t in range(64):
        hh_c = hh.reshape(64, 1)
        ri = jnp.sum(hh_c * whh_i, axis=0, keepdims=True)
        rf = jnp.sum(hh_c * whh_f, axis=0, keepdims=True)
        rg = jnp.sum(hh_c * whh_g, axis=0, keepdims=True)
        ro = jnp.sum(hh_c * whh_o, axis=0, keepdims=True)
        i_t = _sigmoid(gi_s[t:t + 1, :] + ri)
        f_t = _sigmoid(gf_s[t:t + 1, :] + rf)
        o_t = _sigmoid(go_s[t:t + 1, :] + ro)
        g_t = jnp.tanh(gg_s[t:t + 1, :] + rg)
        cc = f_t * cc + i_t * g_t
        hh = o_t * jnp.tanh(cc)

    lane = lax.broadcasted_iota(jnp.int32, (1, 64), 1)
    s_oh = (lane == rti_ref[0]).astype(_F32)
    p_oh = (lane == rti_ref[1]).astype(_F32)
    d_oh = (lane == rti_ref[2]).astype(_F32)
    feat = jnp.concatenate([cc, hh, x_g, s_oh, p_oh, d_oh], axis=1)

    o = jnp.maximum(_dot(feat, w0_ref[...]) + b0_ref[...], 0.0)
    o = jnp.maximum(_dot(o, wa_ref[...]) + ba_ref[...], 0.0)
    o = jnp.maximum(_dot(o, wb_ref[...]) + bb_ref[...], 0.0)
    o = jnp.maximum(_dot(o, wc_ref[...]) + bc_ref[...], 0.0)
    out_ref[...] = o


def kernel(x, edge_index, edge_to_node_index, edge_to_node, routing_table_item,
           hx, cx, W1l, b1, W1r, W2l, b2, W2r, Wih, bih, Whh, bhh,
           W0, b0, Wa, ba, Wb, bb, Wc, bc):
    args = (
        routing_table_item,
        edge_index,
        edge_to_node_index.reshape(1, 64),
        edge_to_node,
        x,
        hx.reshape(1, 64),
        cx.reshape(1, 64),
        W1l, b1.reshape(1, 16), W1r,
        W2l, b2.reshape(1, 64), W2r,
        Wih, bih.reshape(1, 256), Whh, bhh.reshape(1, 256),
        W0, b0.reshape(1, 32), Wa, ba.reshape(1, 16),
        Wb, bb.reshape(1, 8), Wc, bc.reshape(1, 1),
    )
    in_specs = ([pl.BlockSpec(memory_space=pltpu.SMEM)]
                + [pl.BlockSpec(memory_space=pltpu.VMEM)] * (len(args) - 1))
    out = pl.pallas_call(
        _net_body,
        out_shape=jax.ShapeDtypeStruct((1, 1), jnp.float32),
        in_specs=in_specs,
        out_specs=pl.BlockSpec(memory_space=pltpu.VMEM),
        scratch_shapes=[pltpu.VMEM((64, 64), jnp.float32)] * 8,
    )(*args)
    return out.reshape(1)
